# baseline (speedup 1.0000x reference)
import jax
import jax.numpy as jnp
from jax import lax
from jax.experimental import pallas as pl
from jax.experimental.pallas import tpu as pltpu
from jax.experimental.pallas import tpu_sc as plsc

N = 50000
ACC_R = 50176
E = 800000
EP = 819200
EROWS = EP // 128
BN = 2000
GRID = N // BN

_mesh = plsc.VectorSubcoreMesh(core_axis_name="c", subcore_axis_name="s")


def _deg_body(edges, out, dstv, ones_v, zb, wbuf, accd, sem):
    c = lax.axis_index("c")
    s = lax.axis_index("s")

    def fill_ones(i, _):
        ones_v[i, pl.ds(0, 16)] = jnp.ones((16,), jnp.float32)
        return 0

    lax.fori_loop(0, 128, fill_ones, 0)

    def fill_zero(i, _):
        zb[i, pl.ds(0, 16)] = jnp.zeros((16,), jnp.float32)
        return 0

    lax.fori_loop(0, 784, fill_zero, 0)

    def zero_acc(m, _):
        pltpu.sync_copy(zb, accd.at[pl.ds(s * 3136 + m * 784, 784)])
        return 0

    lax.fori_loop(0, 4, zero_acc, 0)
    plsc.subcore_barrier()

    w = s * 2 + c
    base = w * 200

    def step(g, _):
        row0 = base + g * 4
        pltpu.sync_copy(edges.at[1, pl.ds(row0, 4), :], dstv)
        cps = [pltpu.async_copy(ones_v, accd.at[dstv.at[j]], sem, add=True)
               for j in range(4)]
        for cp in cps:
            cp.wait()
        return 0

    lax.fori_loop(0, 50, step, 0)
    plsc.subcore_barrier()

    r0 = s * 3125

    def wb(m, _):
        pltpu.sync_copy(accd.at[pl.ds(r0 + m * 625, 625)], wbuf)
        pltpu.sync_copy(wbuf, out.at[c, pl.ds(r0 + m * 625, 625), :])
        return 0

    lax.fori_loop(0, 5, wb, 0)


_deg_call = pl.kernel(
    _deg_body,
    mesh=_mesh,
    out_type=jax.ShapeDtypeStruct((2, N, 16), jnp.float32),
    scratch_types=[
        pltpu.VMEM((4, 128), jnp.int32),
        pltpu.VMEM((128, 16), jnp.float32),
        pltpu.VMEM((784, 16), jnp.float32),
        pltpu.VMEM((625, 16), jnp.float32),
        pltpu.VMEM_SHARED((ACC_R, 16), jnp.float32),
        pltpu.SemaphoreType.DMA,
    ],
    compiler_params=pltpu.CompilerParams(use_tc_tiling_on_sc=False),
)


def _edge_body(hs, edges, out, srcB, dstB, rows0, rows1, zb, wbuf, accm,
               semA, semB, semS0, semS1):
    c = lax.axis_index("c")
    s = lax.axis_index("s")

    def fill_zero(i, _):
        zb[i, pl.ds(0, 16)] = jnp.zeros((16,), jnp.float32)
        return 0

    lax.fori_loop(0, 392, fill_zero, 0)

    base = s * 400
    r0w = s * 3125

    for p in range(2):
        def zero_acc(m, _):
            pltpu.sync_copy(zb, accm.at[pl.ds(s * 3136 + m * 392, 392)])
            return 0

        lax.fori_loop(0, 8, zero_acc, 0)
        plsc.subcore_barrier()

        off = (2 * c + p) * N

        for h in range(4):
            hb = base + h * 100
            pltpu.sync_copy(edges.at[0, pl.ds(hb, 100), :], srcB)
            pltpu.sync_copy(edges.at[1, pl.ds(hb, 100), :], dstB)

            def adj(r, _):
                for i in range(8):
                    srcB[r, pl.ds(i * 16, 16)] = srcB[r, pl.ds(i * 16, 16)] + off
                return 0

            lax.fori_loop(0, 100, adj, 0)

            def gath(g, buf, sem):
                for j in range(10):
                    pltpu.async_copy(hs.at[srcB.at[g * 10 + j]], buf.at[j], sem)

            def scat(g, buf, sem):
                for j in range(10):
                    pltpu.async_copy(buf.at[j], accm.at[dstB.at[g * 10 + j]],
                                     sem, add=True)

            def drain(sem, buf):
                for j in range(10):
                    pltpu.make_async_copy(hs.at[pl.ds(0, 128)], buf.at[j],
                                          sem).wait()

            gath(0, rows0, semA)

            def pair(g2, _):
                g0 = 2 * g2

                @pl.when(g2 > 0)
                def _():
                    drain(semS1, rows1)

                gath(g0 + 1, rows1, semB)
                drain(semA, rows0)
                scat(g0, rows0, semS0)

                @pl.when(g2 < 4)
                def _():
                    drain(semS0, rows0)
                    gath(g0 + 2, rows0, semA)

                drain(semB, rows1)
                scat(g0 + 1, rows1, semS1)
                return 0

            lax.fori_loop(0, 5, pair, 0)
            drain(semS0, rows0)
            drain(semS1, rows1)

        plsc.subcore_barrier()

        def wb(m, _):
            pltpu.sync_copy(accm.at[pl.ds(r0w + m * 125, 125)], wbuf)
            pltpu.sync_copy(wbuf, out.at[2 * c + p, pl.ds(r0w + m * 125, 125), :])
            return 0

        lax.fori_loop(0, 25, wb, 0)
        plsc.subcore_barrier()


_edge_call = pl.kernel(
    _edge_body,
    mesh=_mesh,
    out_type=jax.ShapeDtypeStruct((4, N, 16), jnp.float32),
    scratch_types=[
        pltpu.VMEM((100, 128), jnp.int32),
        pltpu.VMEM((100, 128), jnp.int32),
        pltpu.VMEM((10, 128, 16), jnp.float32),
        pltpu.VMEM((10, 128, 16), jnp.float32),
        pltpu.VMEM((392, 16), jnp.float32),
        pltpu.VMEM((125, 16), jnp.float32),
        pltpu.VMEM_SHARED((ACC_R, 16), jnp.float32),
        pltpu.SemaphoreType.DMA,
        pltpu.SemaphoreType.DMA,
        pltpu.SemaphoreType.DMA,
        pltpu.SemaphoreType.DMA,
    ],
    compiler_params=pltpu.CompilerParams(use_tc_tiling_on_sc=False),
)


def _fuse_body(t_ref, s_ref, ws_ref, bs_ref, w1a_ref, w1b_ref, h1_ref):
    ps = jnp.maximum(
        jnp.dot(s_ref[...], ws_ref[...], preferred_element_type=jnp.float32)
        + bs_ref[...], 0.0)
    h1_ref[...] = (
        jnp.dot(t_ref[...], w1a_ref[...], preferred_element_type=jnp.float32)
        + jnp.dot(ps, w1b_ref[...], preferred_element_type=jnp.float32))


def _fuse_call(tf, sf, Ws, bs, W1a, W1b):
    return pl.pallas_call(
        _fuse_body,
        grid=(GRID,),
        in_specs=[
            pl.BlockSpec((BN, 64), lambda i: (i, 0)),
            pl.BlockSpec((BN, 128), lambda i: (i, 0)),
            pl.BlockSpec((128, 32), lambda i: (0, 0)),
            pl.BlockSpec((32,), lambda i: (0,)),
            pl.BlockSpec((64, 64), lambda i: (0, 0)),
            pl.BlockSpec((32, 64), lambda i: (0, 0)),
        ],
        out_specs=pl.BlockSpec((BN, 64), lambda i: (i, 0)),
        out_shape=jax.ShapeDtypeStruct((N, 64), jnp.float32),
    )(tf, sf, Ws, bs, W1a, W1b)


def _scale_body(degp_ref, h1_ref, dinv_ref, hs_ref):
    deg = degp_ref[0, :, 0:1] + degp_ref[1, :, 0:1] + 1.0
    dinv = lax.rsqrt(deg)
    dinv_ref[...] = dinv
    hsv = h1_ref[...] * dinv
    for q in range(4):
        hs_ref[q] = hsv[:, 16 * q:16 * (q + 1)]


def _scale_call(degp, h1):
    return pl.pallas_call(
        _scale_body,
        grid=(GRID,),
        in_specs=[
            pl.BlockSpec((2, BN, 16), lambda i: (0, i, 0)),
            pl.BlockSpec((BN, 64), lambda i: (i, 0)),
        ],
        out_specs=[
            pl.BlockSpec((BN, 1), lambda i: (i, 0)),
            pl.BlockSpec((4, BN, 16), lambda i: (0, i, 0)),
        ],
        out_shape=[
            jax.ShapeDtypeStruct((N, 1), jnp.float32),
            jax.ShapeDtypeStruct((4, N, 16), jnp.float32),
        ],
    )(degp, h1)


def _epi1_body(acc_ref, h1_ref, dinv_ref, b1_ref, w2_ref, h2_ref, hs2_ref):
    dinv = dinv_ref[...]
    accc = jnp.concatenate([acc_ref[q] for q in range(4)], axis=1)
    x2 = jnp.maximum(accc * dinv + h1_ref[...] * (dinv * dinv) + b1_ref[...], 0.0)
    h2 = jnp.dot(x2, w2_ref[...], preferred_element_type=jnp.float32)
    h2_ref[...] = h2
    hs2 = h2 * dinv
    for q in range(4):
        hs2_ref[q] = hs2[:, 16 * q:16 * (q + 1)]


def _epi1_call(acc1, h1, dinv, b1, W2):
    return pl.pallas_call(
        _epi1_body,
        grid=(GRID,),
        in_specs=[
            pl.BlockSpec((4, BN, 16), lambda i: (0, i, 0)),
            pl.BlockSpec((BN, 64), lambda i: (i, 0)),
            pl.BlockSpec((BN, 1), lambda i: (i, 0)),
            pl.BlockSpec((64,), lambda i: (0,)),
            pl.BlockSpec((64, 64), lambda i: (0, 0)),
        ],
        out_specs=[
            pl.BlockSpec((BN, 64), lambda i: (i, 0)),
            pl.BlockSpec((4, BN, 16), lambda i: (0, i, 0)),
        ],
        out_shape=[
            jax.ShapeDtypeStruct((N, 64), jnp.float32),
            jax.ShapeDtypeStruct((4, N, 16), jnp.float32),
        ],
    )(acc1, h1, dinv, b1, W2)


def _epi2_body(acc_ref, h2_ref, dinv_ref, b2_ref, wlin_ref, blin_ref, out_ref):
    dinv = dinv_ref[...]
    accc = jnp.concatenate([acc_ref[q] for q in range(4)], axis=1)
    x3 = jnp.maximum(accc * dinv + h2_ref[...] * (dinv * dinv) + b2_ref[...], 0.0)
    out_ref[...] = (
        jnp.dot(x3, wlin_ref[...], preferred_element_type=jnp.float32)
        + blin_ref[...])


def _epi2_call(acc2, h2, dinv, b2, Wlin, blin):
    return pl.pallas_call(
        _epi2_body,
        grid=(GRID,),
        in_specs=[
            pl.BlockSpec((4, BN, 16), lambda i: (0, i, 0)),
            pl.BlockSpec((BN, 64), lambda i: (i, 0)),
            pl.BlockSpec((BN, 1), lambda i: (i, 0)),
            pl.BlockSpec((64,), lambda i: (0,)),
            pl.BlockSpec((64, 1), lambda i: (0, 0)),
            pl.BlockSpec((1,), lambda i: (0,)),
        ],
        out_specs=pl.BlockSpec((BN, 1), lambda i: (i, 0)),
        out_shape=jax.ShapeDtypeStruct((N, 1), jnp.float32),
    )(acc2, h2, dinv, b2, Wlin, blin)


def kernel(temporal_features, static_features, edge_index, Ws, bs, W1, b1, W2,
           b2, Wlin, blin):
    ei = edge_index.astype(jnp.int32)
    pad_src = jnp.zeros((1, EP - E), jnp.int32)
    pad_dst = jnp.full((1, EP - E), N, jnp.int32)
    ei_p = jnp.concatenate([ei, jnp.concatenate([pad_src, pad_dst], axis=0)],
                           axis=1).reshape(2, EROWS, 128)
    W1a, W1b = W1[:64], W1[64:]

    degp = _deg_call(ei_p)
    h1 = _fuse_call(temporal_features, static_features, Ws, bs, W1a, W1b)
    dinv, hs1 = _scale_call(degp, h1)
    acc1 = _edge_call(hs1.reshape(4 * N, 16), ei_p)
    h2, hs2 = _epi1_call(acc1, h1, dinv, b1, W2)
    acc2 = _edge_call(hs2.reshape(4 * N, 16), ei_p)
    outp = _epi2_call(acc2, h2, dinv, b2, Wlin, blin)
    return outp[:, 0]


# speedup vs baseline: 1.1127x; 1.1127x over previous
import jax
import jax.numpy as jnp
from jax import lax
from jax.experimental import pallas as pl
from jax.experimental.pallas import tpu as pltpu
from jax.experimental.pallas import tpu_sc as plsc

N = 50000
ACC_R = 50176
E = 800000
EP = 819200
EROWS = EP // 128
BN = 2000
GRID = N // BN

_mesh = plsc.VectorSubcoreMesh(core_axis_name="c", subcore_axis_name="s")


def _deg_body(edges, out, dstv, ones_v, zb, wbuf, accd, sem):
    c = lax.axis_index("c")
    s = lax.axis_index("s")

    def fill_ones(i, _):
        ones_v[i, pl.ds(0, 16)] = jnp.ones((16,), jnp.float32)
        return 0

    lax.fori_loop(0, 128, fill_ones, 0)

    def fill_zero(i, _):
        zb[i, pl.ds(0, 16)] = jnp.zeros((16,), jnp.float32)
        return 0

    lax.fori_loop(0, 784, fill_zero, 0)

    def zero_acc(m, _):
        pltpu.sync_copy(zb, accd.at[pl.ds(s * 3136 + m * 784, 784)])
        return 0

    lax.fori_loop(0, 4, zero_acc, 0)
    plsc.subcore_barrier()

    w = s * 2 + c
    base = w * 200

    def step(g, _):
        row0 = base + g * 4
        pltpu.sync_copy(edges.at[1, pl.ds(row0, 4), :], dstv)
        cps = [pltpu.async_copy(ones_v, accd.at[dstv.at[j]], sem, add=True)
               for j in range(4)]
        for cp in cps:
            cp.wait()
        return 0

    lax.fori_loop(0, 50, step, 0)
    plsc.subcore_barrier()

    r0 = s * 3125

    def wb(m, _):
        pltpu.sync_copy(accd.at[pl.ds(r0 + m * 625, 625)], wbuf)
        pltpu.sync_copy(wbuf, out.at[c, pl.ds(r0 + m * 625, 625), :])
        return 0

    lax.fori_loop(0, 5, wb, 0)


_deg_call = pl.kernel(
    _deg_body,
    mesh=_mesh,
    out_type=jax.ShapeDtypeStruct((2, N, 16), jnp.float32),
    scratch_types=[
        pltpu.VMEM((4, 128), jnp.int32),
        pltpu.VMEM((128, 16), jnp.float32),
        pltpu.VMEM((784, 16), jnp.float32),
        pltpu.VMEM((625, 16), jnp.float32),
        pltpu.VMEM_SHARED((ACC_R, 16), jnp.float32),
        pltpu.SemaphoreType.DMA,
    ],
    compiler_params=pltpu.CompilerParams(use_tc_tiling_on_sc=False),
)


def _edge_body(hs, edges, out, srcB, dstB, rows0, rows1, zb, wbuf, accm,
               semA, semB, semS0, semS1):
    c = lax.axis_index("c")
    s = lax.axis_index("s")

    def fill_zero(i, _):
        zb[i, pl.ds(0, 16)] = jnp.zeros((16,), jnp.float32)
        zb[i, pl.ds(16, 16)] = jnp.zeros((16,), jnp.float32)
        return 0

    lax.fori_loop(0, 98, fill_zero, 0)

    base = s * 400
    r0w = s * 3125

    def zero_acc(m, _):
        pltpu.sync_copy(zb, accm.at[pl.ds(s * 3136 + m * 98, 98)])
        return 0

    lax.fori_loop(0, 32, zero_acc, 0)
    plsc.subcore_barrier()

    off = c * N

    for h in range(20):
        hb = base + h * 20
        pltpu.sync_copy(edges.at[0, pl.ds(hb, 20), :], srcB)
        pltpu.sync_copy(edges.at[1, pl.ds(hb, 20), :], dstB)

        def adj(r, _):
            for i in range(8):
                srcB[r, pl.ds(i * 16, 16)] = srcB[r, pl.ds(i * 16, 16)] + off
            return 0

        lax.fori_loop(0, 20, adj, 0)

        def gath(g, buf, sem):
            for j in range(2):
                pltpu.async_copy(hs.at[srcB.at[g * 2 + j]], buf.at[j], sem)

        def scat(g, buf, sem):
            for j in range(2):
                pltpu.async_copy(buf.at[j], accm.at[dstB.at[g * 2 + j]],
                                 sem, add=True)

        def drain(sem, buf):
            for j in range(2):
                pltpu.make_async_copy(hs.at[pl.ds(0, 128)], buf.at[j],
                                      sem).wait()

        gath(0, rows0, semA)

        def pair(g2, _):
            g0 = 2 * g2

            @pl.when(g2 > 0)
            def _():
                drain(semS1, rows1)

            gath(g0 + 1, rows1, semB)
            drain(semA, rows0)
            scat(g0, rows0, semS0)

            @pl.when(g2 < 4)
            def _():
                drain(semS0, rows0)
                gath(g0 + 2, rows0, semA)

            drain(semB, rows1)
            scat(g0 + 1, rows1, semS1)
            return 0

        lax.fori_loop(0, 5, pair, 0)
        drain(semS0, rows0)
        drain(semS1, rows1)

    plsc.subcore_barrier()

    def wb(m, _):
        pltpu.sync_copy(accm.at[pl.ds(r0w + m * 125, 125)], wbuf)
        pltpu.sync_copy(wbuf, out.at[c, pl.ds(r0w + m * 125, 125), :])
        return 0

    lax.fori_loop(0, 25, wb, 0)


_edge_call = pl.kernel(
    _edge_body,
    mesh=_mesh,
    out_type=jax.ShapeDtypeStruct((2, N, 32), jnp.float32),
    scratch_types=[
        pltpu.VMEM((20, 128), jnp.int32),
        pltpu.VMEM((20, 128), jnp.int32),
        pltpu.VMEM((2, 128, 32), jnp.float32),
        pltpu.VMEM((2, 128, 32), jnp.float32),
        pltpu.VMEM((98, 32), jnp.float32),
        pltpu.VMEM((125, 32), jnp.float32),
        pltpu.VMEM_SHARED((ACC_R, 32), jnp.float32),
        pltpu.SemaphoreType.DMA,
        pltpu.SemaphoreType.DMA,
        pltpu.SemaphoreType.DMA,
        pltpu.SemaphoreType.DMA,
    ],
    compiler_params=pltpu.CompilerParams(use_tc_tiling_on_sc=False),
)


def _fuse_body(t_ref, s_ref, ws_ref, bs_ref, w1a_ref, w1b_ref, h1_ref):
    ps = jnp.maximum(
        jnp.dot(s_ref[...], ws_ref[...], preferred_element_type=jnp.float32)
        + bs_ref[...], 0.0)
    h1_ref[...] = (
        jnp.dot(t_ref[...], w1a_ref[...], preferred_element_type=jnp.float32)
        + jnp.dot(ps, w1b_ref[...], preferred_element_type=jnp.float32))


def _fuse_call(tf, sf, Ws, bs, W1a, W1b):
    return pl.pallas_call(
        _fuse_body,
        grid=(GRID,),
        in_specs=[
            pl.BlockSpec((BN, 64), lambda i: (i, 0)),
            pl.BlockSpec((BN, 128), lambda i: (i, 0)),
            pl.BlockSpec((128, 32), lambda i: (0, 0)),
            pl.BlockSpec((32,), lambda i: (0,)),
            pl.BlockSpec((64, 64), lambda i: (0, 0)),
            pl.BlockSpec((32, 64), lambda i: (0, 0)),
        ],
        out_specs=pl.BlockSpec((BN, 64), lambda i: (i, 0)),
        out_shape=jax.ShapeDtypeStruct((N, 64), jnp.float32),
    )(tf, sf, Ws, bs, W1a, W1b)


def _scale_body(degp_ref, h1_ref, dinv_ref, hs_ref):
    deg = degp_ref[0, :, 0:1] + degp_ref[1, :, 0:1] + 1.0
    dinv = lax.rsqrt(deg)
    dinv_ref[...] = dinv
    hsv = h1_ref[...] * dinv
    for q in range(2):
        hs_ref[q] = hsv[:, 32 * q:32 * (q + 1)]


def _scale_call(degp, h1):
    return pl.pallas_call(
        _scale_body,
        grid=(GRID,),
        in_specs=[
            pl.BlockSpec((2, BN, 16), lambda i: (0, i, 0)),
            pl.BlockSpec((BN, 64), lambda i: (i, 0)),
        ],
        out_specs=[
            pl.BlockSpec((BN, 1), lambda i: (i, 0)),
            pl.BlockSpec((2, BN, 32), lambda i: (0, i, 0)),
        ],
        out_shape=[
            jax.ShapeDtypeStruct((N, 1), jnp.float32),
            jax.ShapeDtypeStruct((2, N, 32), jnp.float32),
        ],
    )(degp, h1)


def _epi1_body(acc_ref, h1_ref, dinv_ref, b1_ref, w2_ref, h2_ref, hs2_ref):
    dinv = dinv_ref[...]
    accc = jnp.concatenate([acc_ref[q] for q in range(2)], axis=1)
    x2 = jnp.maximum(accc * dinv + h1_ref[...] * (dinv * dinv) + b1_ref[...], 0.0)
    h2 = jnp.dot(x2, w2_ref[...], preferred_element_type=jnp.float32)
    h2_ref[...] = h2
    hs2 = h2 * dinv
    for q in range(2):
        hs2_ref[q] = hs2[:, 32 * q:32 * (q + 1)]


def _epi1_call(acc1, h1, dinv, b1, W2):
    return pl.pallas_call(
        _epi1_body,
        grid=(GRID,),
        in_specs=[
            pl.BlockSpec((2, BN, 32), lambda i: (0, i, 0)),
            pl.BlockSpec((BN, 64), lambda i: (i, 0)),
            pl.BlockSpec((BN, 1), lambda i: (i, 0)),
            pl.BlockSpec((64,), lambda i: (0,)),
            pl.BlockSpec((64, 64), lambda i: (0, 0)),
        ],
        out_specs=[
            pl.BlockSpec((BN, 64), lambda i: (i, 0)),
            pl.BlockSpec((2, BN, 32), lambda i: (0, i, 0)),
        ],
        out_shape=[
            jax.ShapeDtypeStruct((N, 64), jnp.float32),
            jax.ShapeDtypeStruct((2, N, 32), jnp.float32),
        ],
    )(acc1, h1, dinv, b1, W2)


def _epi2_body(acc_ref, h2_ref, dinv_ref, b2_ref, wlin_ref, blin_ref, out_ref):
    dinv = dinv_ref[...]
    accc = jnp.concatenate([acc_ref[q] for q in range(2)], axis=1)
    x3 = jnp.maximum(accc * dinv + h2_ref[...] * (dinv * dinv) + b2_ref[...], 0.0)
    out_ref[...] = (
        jnp.dot(x3, wlin_ref[...], preferred_element_type=jnp.float32)
        + blin_ref[...])


def _epi2_call(acc2, h2, dinv, b2, Wlin, blin):
    return pl.pallas_call(
        _epi2_body,
        grid=(GRID,),
        in_specs=[
            pl.BlockSpec((2, BN, 32), lambda i: (0, i, 0)),
            pl.BlockSpec((BN, 64), lambda i: (i, 0)),
            pl.BlockSpec((BN, 1), lambda i: (i, 0)),
            pl.BlockSpec((64,), lambda i: (0,)),
            pl.BlockSpec((64, 1), lambda i: (0, 0)),
            pl.BlockSpec((1,), lambda i: (0,)),
        ],
        out_specs=pl.BlockSpec((BN, 1), lambda i: (i, 0)),
        out_shape=jax.ShapeDtypeStruct((N, 1), jnp.float32),
    )(acc2, h2, dinv, b2, Wlin, blin)


def kernel(temporal_features, static_features, edge_index, Ws, bs, W1, b1, W2,
           b2, Wlin, blin):
    ei = edge_index.astype(jnp.int32)
    pad_src = jnp.zeros((1, EP - E), jnp.int32)
    pad_dst = jnp.full((1, EP - E), N, jnp.int32)
    ei_p = jnp.concatenate([ei, jnp.concatenate([pad_src, pad_dst], axis=0)],
                           axis=1).reshape(2, EROWS, 128)
    W1a, W1b = W1[:64], W1[64:]

    degp = _deg_call(ei_p)
    h1 = _fuse_call(temporal_features, static_features, Ws, bs, W1a, W1b)
    dinv, hs1 = _scale_call(degp, h1)
    acc1 = _edge_call(hs1.reshape(2 * N, 32), ei_p)
    h2, hs2 = _epi1_call(acc1, h1, dinv, b1, W2)
    acc2 = _edge_call(hs2.reshape(2 * N, 32), ei_p)
    outp = _epi2_call(acc2, h2, dinv, b2, Wlin, blin)
    return outp[:, 0]
